# Initial kernel scaffold; baseline (speedup 1.0000x reference)
import jax
import jax.numpy as jnp
from jax.experimental import pallas as pl


def kernel(world_pos, prev_world_pos, node_type, cells, mesh_pos, params):
    raise NotImplementedError("write your pallas kernel here")


# speedup vs baseline: 1.9271x; 1.9271x over previous
import functools

import jax
import jax.numpy as jnp
from jax import lax
from jax.experimental import pallas as pl
from jax.experimental.pallas import tpu as pltpu
from jax.experimental.pallas import tpu_sc as plsc

N_NODES = 50000
N_CELLS = 100000
NODE_TYPE_SIZE = 9
LATENT = 128
MP_STEPS = 15
OUTPUT_SIZE = 3

NP = 51200
RANGE = 12800
RPT = RANGE // 16
TRASH = RANGE
CAP = 159744
CPT = CAP // 16
NCH = CPT // 128
E = 6 * N_CELLS
EP = 602112
RT = EP // 32
C = 128
KG = 3
NG = RT // (KG * C)
BE = 1024
BN = 1024

_MESH = dict(core_axis_name="c", subcore_axis_name="s", num_cores=2,
             num_subcores=16)


def _dot(x, w):
    return jnp.dot(x.astype(jnp.bfloat16), w,
                   preferred_element_type=jnp.float32)


def _ln(x, g, b):
    mu = jnp.mean(x, axis=-1, keepdims=True)
    var = jnp.mean(x * x, axis=-1, keepdims=True) - mu * mu
    return (x - mu) * lax.rsqrt(var + 1e-5) * g + b


def _gather2_body(tab, ia, ib, oa, ob, rows_v, ia_v, ib_v, sem):
    c = lax.axis_index("c")
    s = lax.axis_index("s")
    base = (c * 16 + s) * RT
    pltpu.sync_copy(ia.at[pl.ds(base, RT)], ia_v)
    pltpu.sync_copy(ib.at[pl.ds(base, RT)], ib_v)

    def run(idx_v, out):
        def body(jg, carry):
            off = jg * (KG * C)
            cps = [
                pltpu.async_copy(tab.at[idx_v.at[pl.ds(off + k * C, C)]],
                                 rows_v.at[pl.ds(k * C, C)], sem)
                for k in range(KG)
            ]
            for cp in cps:
                cp.wait()
            pltpu.sync_copy(rows_v, out.at[pl.ds(base + off, KG * C)])
            return carry
        lax.fori_loop(0, NG, body, 0)

    run(ia_v, oa)
    run(ib_v, ob)


def _make_gather2(v_rows, d):
    mesh = plsc.VectorSubcoreMesh(**_MESH)
    return pl.kernel(
        _gather2_body,
        out_type=(jax.ShapeDtypeStruct((EP, d), jnp.float32),
                  jax.ShapeDtypeStruct((EP, d), jnp.float32)),
        mesh=mesh,
        scratch_types=[
            pltpu.VMEM((KG * C, d), jnp.float32),
            pltpu.VMEM((RT,), jnp.int32),
            pltpu.VMEM((RT,), jnp.int32),
            pltpu.SemaphoreType.DMA,
        ],
    )


def _segsum_body(el, eids, ridx, zz, agg, spm, rows_v, eid_v, idx_v, sem):
    c = lax.axis_index("c")
    t = lax.axis_index("s")
    for r in range(2):
        g = 2 * c + r
        pltpu.sync_copy(zz.at[pl.ds(t * RPT, RPT)],
                        spm.at[pl.ds(t * RPT, RPT)])
        plsc.subcore_barrier()
        tbase = g * CAP + t * CPT

        def body(j, carry):
            off = tbase + j * C
            pltpu.sync_copy(eids.at[pl.ds(off, C)], eid_v)
            pltpu.async_copy(el.at[eid_v], rows_v, sem).wait()
            pltpu.sync_copy(ridx.at[pl.ds(off, C)], idx_v)
            pltpu.sync_copy(rows_v, spm.at[idx_v], add=True)
            return carry

        lax.fori_loop(0, NCH, body, 0)
        plsc.subcore_barrier()
        pltpu.sync_copy(spm.at[pl.ds(t * RPT, RPT)],
                        agg.at[pl.ds(g * RANGE + t * RPT, RPT)])
        plsc.subcore_barrier()


def _make_segsum():
    mesh = plsc.VectorSubcoreMesh(**_MESH)
    return pl.kernel(
        _segsum_body,
        out_type=jax.ShapeDtypeStruct((NP, LATENT), jnp.float32),
        mesh=mesh,
        scratch_types=[
            pltpu.VMEM_SHARED((RANGE + 8, LATENT), jnp.float32),
            pltpu.VMEM((C, LATENT), jnp.float32),
            pltpu.VMEM((C,), jnp.int32),
            pltpu.VMEM((C,), jnp.int32),
            pltpu.SemaphoreType.DMA,
        ],
    )


def _node_enc_k(wp, pwp, nt, w1v, w1o, b1, w2, b2, w3, b3, g, b, out):
    vel = wp[...] - pwp[...]
    oh = (nt[...] == lax.broadcasted_iota(jnp.int32, (BN, NODE_TYPE_SIZE), 1))
    h = jnp.maximum(_dot(vel, w1v[...]) + _dot(oh.astype(jnp.float32), w1o[...])
                    + b1[...], 0.0)
    h = jnp.maximum(_dot(h, w2[...]) + b2[...], 0.0)
    h = _dot(h, w3[...]) + b3[...]
    out[...] = _ln(h, g[...], b[...])


def _edge_enc_k(gs, gr, w1, b1, w2, b2, w3, b3, g, b, out):
    d = gs[...] - gr[...]
    rw = d[:, 0:3]
    rm = d[:, 3:5]
    nw = jnp.sqrt(jnp.sum(rw * rw, axis=1, keepdims=True))
    nm = jnp.sqrt(jnp.sum(rm * rm, axis=1, keepdims=True))
    f = jnp.concatenate([rw, nw, rm, nm, jnp.zeros((BE, 1), jnp.float32)],
                        axis=1)
    h = jnp.maximum(_dot(f, w1[...]) + b1[...], 0.0)
    h = jnp.maximum(_dot(h, w2[...]) + b2[...], 0.0)
    h = _dot(h, w3[...]) + b3[...]
    out[...] = _ln(h, g[...], b[...])


def _edge_blk_k(el, gs, gr, w1a, w1b, w1c, b1, w2, b2, w3, b3, g, b, out):
    x = el[...]
    h = jnp.maximum(_dot(x, w1a[...]) + _dot(gs[...], w1b[...])
                    + _dot(gr[...], w1c[...]) + b1[...], 0.0)
    h = jnp.maximum(_dot(h, w2[...]) + b2[...], 0.0)
    h = _dot(h, w3[...]) + b3[...]
    out[...] = x + _ln(h, g[...], b[...])


def _node_blk_k(nl, agg, w1a, w1b, b1, w2, b2, w3, b3, g, b, out):
    x = nl[...]
    h = jnp.maximum(_dot(x, w1a[...]) + _dot(agg[...], w1b[...]) + b1[...], 0.0)
    h = jnp.maximum(_dot(h, w2[...]) + b2[...], 0.0)
    h = _dot(h, w3[...]) + b3[...]
    out[...] = x + _ln(h, g[...], b[...])


def _dec_k(nl, wp, pwp, w1, b1, w2, b2, w3, b3, out):
    h = jnp.maximum(_dot(nl[...], w1[...]) + b1[...], 0.0)
    h = jnp.maximum(_dot(h, w2[...]) + b2[...], 0.0)
    acc = _dot(h, w3[...]) + b3[...]
    out[...] = 2.0 * wp[...] + acc - pwp[...]


def _prep_mlp(mlp, fold_std=None, fold_mean=None, out_scale=None,
              out_shift=None):
    (w1, b1), (w2, b2), (w3, b3) = mlp
    w1 = jnp.asarray(w1, jnp.float32)
    b1 = jnp.asarray(b1, jnp.float32)
    if fold_std is not None:
        w1 = w1 / fold_std[:, None]
        b1 = b1 - fold_mean @ w1
    w3 = jnp.asarray(w3, jnp.float32)
    b3 = jnp.asarray(b3, jnp.float32)
    if out_scale is not None:
        w3 = w3 * out_scale[None, :]
        b3 = b3 * out_scale + out_shift
    cast = lambda a: jnp.asarray(a, jnp.bfloat16)
    return (cast(w1), b1[None, :], cast(w2),
            jnp.asarray(b2, jnp.float32)[None, :], cast(w3), b3[None, :])


def kernel(world_pos, prev_world_pos, node_type, cells, mesh_pos, params):
    f32 = jnp.float32
    s0 = jnp.concatenate([cells[:, 0], cells[:, 1], cells[:, 2]])
    r0 = jnp.concatenate([cells[:, 1], cells[:, 2], cells[:, 0]])
    senders = jnp.concatenate([s0, r0]).astype(jnp.int32)
    receivers = jnp.concatenate([r0, s0]).astype(jnp.int32)
    perm = jnp.argsort(receivers)
    rs = receivers[perm]
    ss = senders[perm]
    pad = EP - E
    ss_g = jnp.concatenate([ss, jnp.zeros((pad,), jnp.int32)])
    rs_g = jnp.concatenate([rs, jnp.zeros((pad,), jnp.int32)])
    rs_p = jnp.concatenate([rs, jnp.full((pad,), NP, jnp.int32)])
    rb = jnp.searchsorted(rs_p, jnp.arange(5, dtype=jnp.int32) * RANGE)
    rb = rb.astype(jnp.int32)
    k = jnp.arange(CAP, dtype=jnp.int32)
    eid2 = rb[:4, None] + k[None, :]
    valid = eid2 < rb[1:, None]
    eid2 = jnp.where(valid, eid2, EP - 1)
    rvals = rs_p[eid2]
    ridx2 = jnp.where(
        valid, rvals - jnp.arange(4, dtype=jnp.int32)[:, None] * RANGE, TRASH)
    eids_flat = eid2.reshape(-1)
    ridx_flat = ridx2.reshape(-1).astype(jnp.int32)
    zeros_hbm = jnp.zeros((RANGE, LATENT), f32)

    def padn(x, val=0):
        return jnp.concatenate(
            [x, jnp.full((NP - N_NODES,) + x.shape[1:], val, x.dtype)])

    wp_p = padn(world_pos)
    pwp_p = padn(prev_world_pos)
    nt_p = padn(node_type.astype(jnp.int32))
    wpm = jnp.concatenate(
        [world_pos, mesh_pos, jnp.zeros((N_NODES, 123), f32)], axis=1)

    p = params
    nn_std = jnp.asarray(p['node_norm']['std'], f32)
    nn_mean = jnp.asarray(p['node_norm']['mean'], f32)
    en_std = jnp.asarray(p['edge_norm']['std'], f32)
    en_mean = jnp.asarray(p['edge_norm']['mean'], f32)
    on_std = jnp.asarray(p['output_norm']['std'], f32)
    on_mean = jnp.asarray(p['output_norm']['mean'], f32)

    ne_w1, ne_b1, ne_w2, ne_b2, ne_w3, ne_b3 = _prep_mlp(
        p['node_enc']['mlp'], fold_std=nn_std, fold_mean=nn_mean)
    ne_w1v, ne_w1o = ne_w1[:3], ne_w1[3:]
    ne_g, ne_b = (jnp.asarray(a, f32)[None, :] for a in p['node_enc']['ln'])

    ee_w1, ee_b1, ee_w2, ee_b2, ee_w3, ee_b3 = _prep_mlp(
        p['edge_enc']['mlp'], fold_std=en_std, fold_mean=en_mean)
    ee_w1 = jnp.concatenate(
        [ee_w1, jnp.zeros((1, LATENT), jnp.bfloat16)])
    ee_g, ee_b = (jnp.asarray(a, f32)[None, :] for a in p['edge_enc']['ln'])

    def stack(getter):
        return jnp.stack([getter(blk) for blk in p['blocks']])

    bf = jnp.bfloat16
    ew = {
        'w1a': stack(lambda k: jnp.asarray(k['edge']['mlp'][0][0][:128], bf)),
        'w1b': stack(lambda k: jnp.asarray(k['edge']['mlp'][0][0][128:256], bf)),
        'w1c': stack(lambda k: jnp.asarray(k['edge']['mlp'][0][0][256:], bf)),
        'b1': stack(lambda k: k['edge']['mlp'][0][1][None, :]),
        'w2': stack(lambda k: jnp.asarray(k['edge']['mlp'][1][0], bf)),
        'b2': stack(lambda k: k['edge']['mlp'][1][1][None, :]),
        'w3': stack(lambda k: jnp.asarray(k['edge']['mlp'][2][0], bf)),
        'b3': stack(lambda k: k['edge']['mlp'][2][1][None, :]),
        'g': stack(lambda k: k['edge']['ln'][0][None, :]),
        'b': stack(lambda k: k['edge']['ln'][1][None, :]),
    }
    nw = {
        'w1a': stack(lambda k: jnp.asarray(k['node']['mlp'][0][0][:128], bf)),
        'w1b': stack(lambda k: jnp.asarray(k['node']['mlp'][0][0][128:], bf)),
        'b1': stack(lambda k: k['node']['mlp'][0][1][None, :]),
        'w2': stack(lambda k: jnp.asarray(k['node']['mlp'][1][0], bf)),
        'b2': stack(lambda k: k['node']['mlp'][1][1][None, :]),
        'w3': stack(lambda k: jnp.asarray(k['node']['mlp'][2][0], bf)),
        'b3': stack(lambda k: k['node']['mlp'][2][1][None, :]),
        'g': stack(lambda k: k['node']['ln'][0][None, :]),
        'b': stack(lambda k: k['node']['ln'][1][None, :]),
    }
    de_w1, de_b1, de_w2, de_b2, de_w3, de_b3 = _prep_mlp(
        p['dec']['mlp'], out_scale=on_std, out_shift=on_mean)

    gEN = NP // BN
    gEE = EP // BE
    W = pl.BlockSpec(index_map=lambda i: (0, 0))

    node_enc = pl.pallas_call(
        _node_enc_k, grid=(gEN,),
        in_specs=[pl.BlockSpec((BN, 3), lambda i: (i, 0)),
                  pl.BlockSpec((BN, 3), lambda i: (i, 0)),
                  pl.BlockSpec((BN, 1), lambda i: (i, 0))] + [W] * 9,
        out_specs=pl.BlockSpec((BN, LATENT), lambda i: (i, 0)),
        out_shape=jax.ShapeDtypeStruct((NP, LATENT), f32))

    edge_enc = pl.pallas_call(
        _edge_enc_k, grid=(gEE,),
        in_specs=[pl.BlockSpec((BE, LATENT), lambda i: (i, 0)),
                  pl.BlockSpec((BE, LATENT), lambda i: (i, 0))] + [W] * 8,
        out_specs=pl.BlockSpec((BE, LATENT), lambda i: (i, 0)),
        out_shape=jax.ShapeDtypeStruct((EP, LATENT), f32))

    edge_blk = pl.pallas_call(
        _edge_blk_k, grid=(gEE,),
        in_specs=[pl.BlockSpec((BE, LATENT), lambda i: (i, 0))] * 3 + [W] * 10,
        out_specs=pl.BlockSpec((BE, LATENT), lambda i: (i, 0)),
        out_shape=jax.ShapeDtypeStruct((EP, LATENT), f32))

    node_blk = pl.pallas_call(
        _node_blk_k, grid=(gEN,),
        in_specs=[pl.BlockSpec((BN, LATENT), lambda i: (i, 0))] * 2 + [W] * 9,
        out_specs=pl.BlockSpec((BN, LATENT), lambda i: (i, 0)),
        out_shape=jax.ShapeDtypeStruct((NP, LATENT), f32))

    dec = pl.pallas_call(
        _dec_k, grid=(gEN,),
        in_specs=[pl.BlockSpec((BN, LATENT), lambda i: (i, 0)),
                  pl.BlockSpec((BN, 3), lambda i: (i, 0)),
                  pl.BlockSpec((BN, 3), lambda i: (i, 0))] + [W] * 6,
        out_specs=pl.BlockSpec((BN, 3), lambda i: (i, 0)),
        out_shape=jax.ShapeDtypeStruct((NP, 3), f32))

    gather_feat = _make_gather2(N_NODES, LATENT)
    gather_lat = _make_gather2(NP, LATENT)
    segsum = _make_segsum()

    node_lat = node_enc(wp_p, pwp_p, nt_p, ne_w1v, ne_w1o, ne_b1, ne_w2,
                        ne_b2, ne_w3, ne_b3, ne_g, ne_b)
    gs8, gr8 = gather_feat(wpm, ss_g, rs_g)
    edge_lat = edge_enc(gs8, gr8, ee_w1, ee_b1, ee_w2, ee_b2, ee_w3, ee_b3,
                        ee_g, ee_b)

    def step(carry, wts):
        e, n = wts
        node_lat, edge_lat = carry
        gs, gr = gather_lat(node_lat, ss_g, rs_g)
        edge_lat = edge_blk(edge_lat, gs, gr, e['w1a'], e['w1b'], e['w1c'],
                            e['b1'], e['w2'], e['b2'], e['w3'], e['b3'],
                            e['g'], e['b'])
        agg = segsum(edge_lat, eids_flat, ridx_flat, zeros_hbm)
        node_lat = node_blk(node_lat, agg, n['w1a'], n['w1b'], n['b1'],
                            n['w2'], n['b2'], n['w3'], n['b3'], n['g'],
                            n['b'])
        return (node_lat, edge_lat), None

    (node_lat, edge_lat), _ = lax.scan(step, (node_lat, edge_lat), (ew, nw))

    pos = dec(node_lat, wp_p, pwp_p, de_w1, de_b1, de_w2, de_b2, de_w3,
              de_b3)
    return pos[:N_NODES]
